# baseline (speedup 1.0000x reference)
import functools

import jax
import jax.numpy as jnp
from jax import lax
from jax.experimental import pallas as pl
from jax.experimental.pallas import tpu as pltpu
from jax.experimental.pallas import tpu_sc as plsc

N = 524288
R0 = N // 128
R1 = R0 // 128
NRAYS = 8192
NWORKERS = 32
CHUNK = N // NWORKERS
L = 16


def _seq_scan_sublane(ref, nrows):
    def body(c, _):
        ref[pl.ds(c, 1), :] = ref[pl.ds(c, 1), :] + ref[pl.ds(c - 1, 1), :]
        return 0
    lax.fori_loop(1, nrows, body, 0, unroll=False)


CB = 512
GRID = R0 // CB


def _elem_kernel(dt_ref, sdf_ref, dirs_ref, grads_ref, car_ref, beta_ref,
                 eps_ref, alpha_ref, logst_ref):
    car = car_ref[0]
    beta = beta_ref[0]
    eps = eps_ref[0]
    x, y, z = dirs_ref[0], dirs_ref[1], dirs_ref[2]
    gx, gy, gz = grads_ref[0], grads_ref[1], grads_ref[2]
    true_cos = (x * gx + z * gz) + y * gy
    relu = lambda v: jnp.maximum(v, 0.0)
    iter_cos = relu(-true_cos * 0.5 + 0.5) * (1.0 - car) + relu(-true_cos) * car
    iter_cos = -iter_cos
    dt = dt_ref[...]
    sdf = sdf_ref[...]
    est_next = sdf + iter_cos * dt * 0.5
    est_prev = sdf - iter_cos * dt * 0.5
    prev_cdf = jax.nn.sigmoid(est_prev * beta)
    next_cdf = jax.nn.sigmoid(est_next * beta)
    alpha = jnp.clip((prev_cdf - next_cdf + eps) / (prev_cdf + eps), 0.0, 1.0)
    alpha_ref[...] = alpha
    logs = jnp.log(jnp.float32(1.00000095367431640625) - alpha)
    logst_ref[...] = logs.T


def _elem(dt2, sdf2, dirs_t, grads_t, car, beta, interpret=False):
    return pl.pallas_call(
        _elem_kernel,
        out_shape=(
            jax.ShapeDtypeStruct((R0, 128), jnp.float32),
            jax.ShapeDtypeStruct((128, R0), jnp.float32),
        ),
        grid=(GRID,),
        in_specs=[
            pl.BlockSpec((CB, 128), lambda i: (i, 0)),
            pl.BlockSpec((CB, 128), lambda i: (i, 0)),
            pl.BlockSpec((3, CB, 128), lambda i: (0, i, 0)),
            pl.BlockSpec((3, CB, 128), lambda i: (0, i, 0)),
            pl.BlockSpec(memory_space=pltpu.SMEM),
            pl.BlockSpec(memory_space=pltpu.SMEM),
            pl.BlockSpec(memory_space=pltpu.SMEM),
        ],
        out_specs=(
            pl.BlockSpec((CB, 128), lambda i: (i, 0)),
            pl.BlockSpec((128, CB), lambda i: (0, i)),
        ),
        interpret=interpret,
    )(dt2, sdf2, dirs_t, grads_t, car, beta,
      jnp.full((1,), 1e-06, dtype=jnp.float32))


def _scan_kernel(lt_ref, exclt_ref, s0_ref, s1_ref, s2_ref):
    lt = lt_ref[...]
    s0_ref[...] = lt
    _seq_scan_sublane(s0_ref, 128)
    lasts0 = s0_ref[pl.ds(127, 1), :]
    s1_ref[...] = lasts0.reshape(R1, 128).T
    _seq_scan_sublane(s1_ref, 128)
    lasts1 = s1_ref[pl.ds(127, 1), :]
    s2_ref[...] = lasts1.reshape(R1, 1)
    _seq_scan_sublane(s2_ref, R1)
    inc2 = s2_ref[...]
    off2 = jnp.concatenate([jnp.zeros((1, 1), jnp.float32), inc2[:-1, :]],
                           axis=0).reshape(1, R1)
    lvl1_t = s1_ref[...] + off2
    lvl1 = lvl1_t.T.reshape(1, R0)
    off0 = jnp.concatenate([jnp.zeros((1, 1), jnp.float32), lvl1[:, :-1]],
                           axis=1)
    inc_t = s0_ref[...] + off0
    exclt_ref[...] = inc_t - lt


def _scan(logs_t, interpret=False):
    return pl.pallas_call(
        _scan_kernel,
        out_shape=jax.ShapeDtypeStruct((128, R0), jnp.float32),
        scratch_shapes=[
            pltpu.VMEM((128, R0), jnp.float32),
            pltpu.VMEM((128, R1), jnp.float32),
            pltpu.VMEM((R1, 1), jnp.float32),
        ],
        interpret=interpret,
    )(logs_t)


def _finish_kernel(exclt_ref, base_ref, alpha_ref, t_ref, w_ref):
    within = exclt_ref[...].T - base_ref[...]
    t = jnp.exp(within)
    t_ref[...] = t
    w_ref[...] = alpha_ref[...] * t


def _finish(excl_t, base2, alpha2, interpret=False):
    spec = pl.BlockSpec((CB, 128), lambda i: (i, 0))
    tspec = pl.BlockSpec((128, CB), lambda i: (0, i))
    return pl.pallas_call(
        _finish_kernel,
        out_shape=(
            jax.ShapeDtypeStruct((R0, 128), jnp.float32),
            jax.ShapeDtypeStruct((R0, 128), jnp.float32),
        ),
        grid=(GRID,),
        in_specs=[tspec, spec, spec],
        out_specs=(spec, spec),
        interpret=interpret,
    )(excl_t, base2, alpha2)


def _expand_body(excl_hbm, cu_hbm, base_hbm,
                 cu_v, ft_v, g_v, z_v, zm_v, ic_v, base_v, sem_g):
    wid = lax.axis_index("s") * 2 + lax.axis_index("c")
    lo = wid * CHUNK
    nblk = CHUNK // L

    pltpu.sync_copy(cu_hbm, cu_v)

    zeros16 = jnp.zeros((L,), jnp.int32)

    def zbody(k, _):
        z_v[pl.ds(k * L, L)] = zeros16
        return 0

    lax.fori_loop(0, nblk, zbody, 0, unroll=4)

    def tbody(j, _):
        cu16 = cu_v[pl.ds(j * L, L)]
        ft_v[pl.ds(j * L, L)] = (cu16 & 127) * R0 + (cu16 >> 7)
        return 0

    lax.fori_loop(0, NRAYS // L, tbody, 0, unroll=8)

    copies = [
        pltpu.async_copy(excl_hbm.at[ft_v.at[pl.ds(j * 128, 128)]],
                         g_v.at[pl.ds(j * 128, 128)], sem_g)
        for j in range(NRAYS // 128)
    ]

    iota16 = lax.iota(jnp.int32, L)

    def sbody(j, cnt):
        cu16 = cu_v[pl.ds(j * L, L)]
        cnt = cnt + jnp.where(cu16 <= lo, 1, 0).astype(jnp.int32)
        idx16 = cu16 - lo
        rid16 = iota16 + j * L
        mask = (idx16 >= 0) & (idx16 < CHUNK)
        plsc.store_scatter(z_v, [idx16], rid16, mask=mask)
        return cnt

    cnt = lax.fori_loop(0, NRAYS // L, sbody, jnp.zeros((L,), jnp.int32),
                        unroll=4)
    r_lo = jnp.broadcast_to(jnp.sum(cnt) - 1, (L,))

    def abody(k, _):
        zm_v[pl.ds(k * L, L)] = plsc.cummax(z_v[pl.ds(k * L, L)])
        return 0

    lax.fori_loop(0, nblk, abody, 0, unroll=4)

    ic_v[pl.ds(0, L)] = r_lo

    def bbody(kk, carry):
        idx = (iota16 + kk * L) * L + (L - 1)
        bm16 = plsc.load_gather(zm_v, [idx])
        c16 = jnp.maximum(plsc.cummax(bm16), carry)
        ic_v[pl.ds(L + kk * L, L)] = c16
        return jnp.broadcast_to(jnp.max(c16), (L,))

    lax.fori_loop(0, nblk // L, bbody, r_lo, unroll=1)

    for c in copies:
        c.wait()

    def cbody(k, _):
        m = zm_v[pl.ds(k * L, L)]
        prev = plsc.load_gather(ic_v, [jnp.broadcast_to(k + L - 1, (L,))])
        base_v[pl.ds(k * L, L)] = plsc.load_gather(g_v, [jnp.maximum(m, prev)])
        return 0

    lax.fori_loop(0, nblk, cbody, 0, unroll=4)

    pltpu.sync_copy(base_v, base_hbm.at[pl.ds(lo, CHUNK)])


def _base_from_starts(excl_flat, cu_seqlens):
    cu_starts = lax.slice(cu_seqlens, (0,), (NRAYS,))
    return pl.kernel(
        _expand_body,
        out_type=jax.ShapeDtypeStruct((N,), jnp.float32),
        mesh=plsc.VectorSubcoreMesh(core_axis_name="c", subcore_axis_name="s"),
        compiler_params=pltpu.CompilerParams(needs_layout_passes=False),
        scratch_types=[
            pltpu.VMEM((NRAYS,), jnp.int32),
            pltpu.VMEM((NRAYS,), jnp.int32),
            pltpu.VMEM((NRAYS,), jnp.float32),
            pltpu.VMEM((CHUNK,), jnp.int32),
            pltpu.VMEM((CHUNK,), jnp.int32),
            pltpu.VMEM((CHUNK // L + L,), jnp.int32),
            pltpu.VMEM((CHUNK,), jnp.float32),
            pltpu.SemaphoreType.DMA,
        ],
    )(excl_flat, cu_starts)


def kernel(samples_dt, samples_dirs, sdf, gradients, cos_anneal_ratio,
           logistic_beta, cu_seqlens, interpret=False):
    dt2 = samples_dt.reshape(R0, 128)
    sdf2 = sdf.reshape(R0, 128)
    dirs_t = samples_dirs.T.reshape(3, R0, 128)
    grads_t = gradients.T.reshape(3, R0, 128)
    alpha2, logs_t = _elem(dt2, sdf2, dirs_t, grads_t,
                           cos_anneal_ratio, logistic_beta,
                           interpret=interpret)
    excl_t = _scan(logs_t, interpret=interpret)
    base2 = _base_from_starts(excl_t.reshape(N), cu_seqlens).reshape(R0, 128)
    t2, w2 = _finish(excl_t, base2, alpha2, interpret=interpret)
    return (w2.reshape(N, 1), t2.reshape(N, 1), alpha2.reshape(N, 1))


# speedup vs baseline: 1.1621x; 1.1621x over previous
import functools

import jax
import jax.numpy as jnp
from jax import lax
from jax.experimental import pallas as pl
from jax.experimental.pallas import tpu as pltpu
from jax.experimental.pallas import tpu_sc as plsc

N = 524288
R0 = N // 128
R1 = R0 // 128
NRAYS = 8192
NWORKERS = 32
CHUNK = N // NWORKERS
L = 16


def _seq_scan_sublane(ref, nrows):
    def body(c, _):
        ref[pl.ds(c, 1), :] = ref[pl.ds(c, 1), :] + ref[pl.ds(c - 1, 1), :]
        return 0
    lax.fori_loop(1, nrows, body, 0, unroll=False)


def _alpha_scan_kernel(dt_ref, sdf_ref, dirs_ref, grads_ref, car_ref, beta_ref,
                       eps_ref, alpha_ref, excl_ref, s0_ref, s1_ref, s2_ref):
    car = car_ref[0]
    beta = beta_ref[0]
    eps = eps_ref[0]
    x, y, z = dirs_ref[0], dirs_ref[1], dirs_ref[2]
    gx, gy, gz = grads_ref[0], grads_ref[1], grads_ref[2]
    true_cos = (x * gx + z * gz) + y * gy
    relu = lambda v: jnp.maximum(v, 0.0)
    iter_cos = relu(-true_cos * 0.5 + 0.5) * (1.0 - car) + relu(-true_cos) * car
    iter_cos = -iter_cos
    dt = dt_ref[...]
    sdf = sdf_ref[...]
    est_next = sdf + iter_cos * dt * 0.5
    est_prev = sdf - iter_cos * dt * 0.5
    prev_cdf = jax.nn.sigmoid(est_prev * beta)
    next_cdf = jax.nn.sigmoid(est_next * beta)
    alpha = jnp.clip((prev_cdf - next_cdf + eps) / (prev_cdf + eps), 0.0, 1.0)
    alpha_ref[...] = alpha
    logs = jnp.log(jnp.float32(1.00000095367431640625) - alpha)

    lt = logs.T
    s0_ref[...] = lt
    _seq_scan_sublane(s0_ref, 128)
    lasts0 = s0_ref[pl.ds(127, 1), :]
    s1_ref[...] = lasts0.reshape(R1, 128).T
    _seq_scan_sublane(s1_ref, 128)
    lasts1 = s1_ref[pl.ds(127, 1), :]
    s2_ref[...] = lasts1.reshape(R1, 1)
    _seq_scan_sublane(s2_ref, R1)
    inc2 = s2_ref[...]
    off2 = jnp.concatenate([jnp.zeros((1, 1), jnp.float32), inc2[:-1, :]],
                           axis=0).reshape(1, R1)
    lvl1_t = s1_ref[...] + off2
    lvl1 = lvl1_t.T.reshape(1, R0)
    off0 = jnp.concatenate([jnp.zeros((1, 1), jnp.float32), lvl1[:, :-1]],
                           axis=1)
    inc_t = s0_ref[...] + off0
    excl_t = inc_t - lt
    excl_ref[...] = excl_t.T


def _alpha_scan(dt2, sdf2, dirs_t, grads_t, car, beta, interpret=False):
    return pl.pallas_call(
        _alpha_scan_kernel,
        out_shape=(
            jax.ShapeDtypeStruct((R0, 128), jnp.float32),
            jax.ShapeDtypeStruct((R0, 128), jnp.float32),
        ),
        in_specs=[
            pl.BlockSpec(memory_space=pltpu.VMEM),
            pl.BlockSpec(memory_space=pltpu.VMEM),
            pl.BlockSpec(memory_space=pltpu.VMEM),
            pl.BlockSpec(memory_space=pltpu.VMEM),
            pl.BlockSpec(memory_space=pltpu.SMEM),
            pl.BlockSpec(memory_space=pltpu.SMEM),
            pl.BlockSpec(memory_space=pltpu.SMEM),
        ],
        out_specs=(
            pl.BlockSpec(memory_space=pltpu.VMEM),
            pl.BlockSpec(memory_space=pltpu.VMEM),
        ),
        scratch_shapes=[
            pltpu.VMEM((128, R0), jnp.float32),
            pltpu.VMEM((128, R1), jnp.float32),
            pltpu.VMEM((R1, 1), jnp.float32),
        ],
        interpret=interpret,
    )(dt2, sdf2, dirs_t, grads_t, car, beta,
      jnp.full((1,), 1e-06, dtype=jnp.float32))


def _finish_kernel(excl_ref, base_ref, alpha_ref, t_ref, w_ref):
    within = excl_ref[...] - base_ref[...]
    t = jnp.exp(within)
    t_ref[...] = t
    w_ref[...] = alpha_ref[...] * t


def _finish(excl2, base2, alpha2, interpret=False):
    spec = pl.BlockSpec((512, 128), lambda i: (i, 0))
    return pl.pallas_call(
        _finish_kernel,
        out_shape=(
            jax.ShapeDtypeStruct((R0, 128), jnp.float32),
            jax.ShapeDtypeStruct((R0, 128), jnp.float32),
        ),
        grid=(R0 // 512,),
        in_specs=[spec, spec, spec],
        out_specs=(spec, spec),
        interpret=interpret,
    )(excl2, base2, alpha2)


def _expand_body(excl_hbm, cu_hbm, base_hbm,
                 cu_v, g_v, z_v, zm_v, ic_v, base_v, sem_g):
    wid = lax.axis_index("s") * 2 + lax.axis_index("c")
    lo = wid * CHUNK
    nblk = CHUNK // L

    pltpu.sync_copy(cu_hbm, cu_v)

    zeros16 = jnp.zeros((L,), jnp.int32)

    def zbody(k, _):
        z_v[pl.ds(k * L, L)] = zeros16
        return 0

    lax.fori_loop(0, nblk, zbody, 0, unroll=4)

    copies = [
        pltpu.async_copy(excl_hbm.at[cu_v.at[pl.ds(j * 128, 128)]],
                         g_v.at[pl.ds(j * 128, 128)], sem_g)
        for j in range(NRAYS // 128)
    ]

    iota16 = lax.iota(jnp.int32, L)

    def sbody(j, cnt):
        cu16 = cu_v[pl.ds(j * L, L)]
        cnt = cnt + jnp.where(cu16 <= lo, 1, 0).astype(jnp.int32)
        idx16 = cu16 - lo
        rid16 = iota16 + j * L
        mask = (idx16 >= 0) & (idx16 < CHUNK)
        plsc.store_scatter(z_v, [idx16], rid16, mask=mask)
        return cnt

    cnt = lax.fori_loop(0, NRAYS // L, sbody, jnp.zeros((L,), jnp.int32),
                        unroll=4)
    r_lo = jnp.broadcast_to(jnp.sum(cnt) - 1, (L,))

    def abody(k, _):
        zm_v[pl.ds(k * L, L)] = plsc.cummax(z_v[pl.ds(k * L, L)])
        return 0

    lax.fori_loop(0, nblk, abody, 0, unroll=1)

    ic_v[pl.ds(0, L)] = r_lo

    def bbody(kk, carry):
        idx = (iota16 + kk * L) * L + (L - 1)
        bm16 = plsc.load_gather(zm_v, [idx])
        c16 = jnp.maximum(plsc.cummax(bm16), carry)
        ic_v[pl.ds(L + kk * L, L)] = c16
        return jnp.broadcast_to(jnp.max(c16), (L,))

    lax.fori_loop(0, nblk // L, bbody, r_lo, unroll=1)

    for c in copies:
        c.wait()

    def cbody(k, _):
        m = zm_v[pl.ds(k * L, L)]
        prev = plsc.load_gather(ic_v, [jnp.broadcast_to(k + L - 1, (L,))])
        base_v[pl.ds(k * L, L)] = plsc.load_gather(g_v, [jnp.maximum(m, prev)])
        return 0

    lax.fori_loop(0, nblk, cbody, 0, unroll=1)

    pltpu.sync_copy(base_v, base_hbm.at[pl.ds(lo, CHUNK)])


def _base_from_starts(excl_flat, cu_seqlens):
    cu_starts = lax.slice(cu_seqlens, (0,), (NRAYS,))
    return pl.kernel(
        _expand_body,
        out_type=jax.ShapeDtypeStruct((N,), jnp.float32),
        mesh=plsc.VectorSubcoreMesh(core_axis_name="c", subcore_axis_name="s"),
        compiler_params=pltpu.CompilerParams(needs_layout_passes=False),
        scratch_types=[
            pltpu.VMEM((NRAYS,), jnp.int32),
            pltpu.VMEM((NRAYS,), jnp.float32),
            pltpu.VMEM((CHUNK,), jnp.int32),
            pltpu.VMEM((CHUNK,), jnp.int32),
            pltpu.VMEM((CHUNK // L + L,), jnp.int32),
            pltpu.VMEM((CHUNK,), jnp.float32),
            pltpu.SemaphoreType.DMA,
        ],
    )(excl_flat, cu_starts)


def kernel(samples_dt, samples_dirs, sdf, gradients, cos_anneal_ratio,
           logistic_beta, cu_seqlens, interpret=False):
    dt2 = samples_dt.reshape(R0, 128)
    sdf2 = sdf.reshape(R0, 128)
    dirs_t = samples_dirs.T.reshape(3, R0, 128)
    grads_t = gradients.T.reshape(3, R0, 128)
    alpha2, excl2 = _alpha_scan(dt2, sdf2, dirs_t, grads_t,
                                cos_anneal_ratio, logistic_beta,
                                interpret=interpret)
    base2 = _base_from_starts(excl2.reshape(N), cu_seqlens).reshape(R0, 128)
    t2, w2 = _finish(excl2, base2, alpha2, interpret=interpret)
    return (w2.reshape(N, 1), t2.reshape(N, 1), alpha2.reshape(N, 1))


# speedup vs baseline: 1.1633x; 1.0011x over previous
import functools

import jax
import jax.numpy as jnp
from jax import lax
from jax.experimental import pallas as pl
from jax.experimental.pallas import tpu as pltpu
from jax.experimental.pallas import tpu_sc as plsc

N = 524288
R0 = N // 128
R1 = R0 // 128
NRAYS = 8192
NWORKERS = 32
CHUNK = N // NWORKERS
L = 16


def _seq_scan_sublane(ref, nrows):
    def body(c, _):
        ref[pl.ds(c, 1), :] = ref[pl.ds(c, 1), :] + ref[pl.ds(c - 1, 1), :]
        return 0
    lax.fori_loop(1, nrows, body, 0, unroll=4)


def _alpha_scan_kernel(dt_ref, sdf_ref, dirs_ref, grads_ref, car_ref, beta_ref,
                       eps_ref, alpha_ref, excl_ref, s0_ref, s1_ref, s2_ref):
    car = car_ref[0]
    beta = beta_ref[0]
    eps = eps_ref[0]
    x, y, z = dirs_ref[0], dirs_ref[1], dirs_ref[2]
    gx, gy, gz = grads_ref[0], grads_ref[1], grads_ref[2]
    true_cos = (x * gx + z * gz) + y * gy
    relu = lambda v: jnp.maximum(v, 0.0)
    iter_cos = relu(-true_cos * 0.5 + 0.5) * (1.0 - car) + relu(-true_cos) * car
    iter_cos = -iter_cos
    dt = dt_ref[...]
    sdf = sdf_ref[...]
    est_next = sdf + iter_cos * dt * 0.5
    est_prev = sdf - iter_cos * dt * 0.5
    prev_cdf = jax.nn.sigmoid(est_prev * beta)
    next_cdf = jax.nn.sigmoid(est_next * beta)
    alpha = jnp.clip((prev_cdf - next_cdf + eps) / (prev_cdf + eps), 0.0, 1.0)
    alpha_ref[...] = alpha
    logs = jnp.log(jnp.float32(1.00000095367431640625) - alpha)

    lt = logs.T
    s0_ref[...] = lt
    _seq_scan_sublane(s0_ref, 128)
    lasts0 = s0_ref[pl.ds(127, 1), :]
    s1_ref[...] = lasts0.reshape(R1, 128).T
    _seq_scan_sublane(s1_ref, 128)
    lasts1 = s1_ref[pl.ds(127, 1), :]
    s2_ref[...] = lasts1.reshape(R1, 1)
    _seq_scan_sublane(s2_ref, R1)
    inc2 = s2_ref[...]
    off2 = jnp.concatenate([jnp.zeros((1, 1), jnp.float32), inc2[:-1, :]],
                           axis=0).reshape(1, R1)
    lvl1_t = s1_ref[...] + off2
    lvl1 = lvl1_t.T.reshape(1, R0)
    off0 = jnp.concatenate([jnp.zeros((1, 1), jnp.float32), lvl1[:, :-1]],
                           axis=1)
    inc_t = s0_ref[...] + off0
    excl_t = inc_t - lt
    excl_ref[...] = excl_t.T


def _alpha_scan(dt2, sdf2, dirs_t, grads_t, car, beta, interpret=False):
    return pl.pallas_call(
        _alpha_scan_kernel,
        out_shape=(
            jax.ShapeDtypeStruct((R0, 128), jnp.float32),
            jax.ShapeDtypeStruct((R0, 128), jnp.float32),
        ),
        in_specs=[
            pl.BlockSpec(memory_space=pltpu.VMEM),
            pl.BlockSpec(memory_space=pltpu.VMEM),
            pl.BlockSpec(memory_space=pltpu.VMEM),
            pl.BlockSpec(memory_space=pltpu.VMEM),
            pl.BlockSpec(memory_space=pltpu.SMEM),
            pl.BlockSpec(memory_space=pltpu.SMEM),
            pl.BlockSpec(memory_space=pltpu.SMEM),
        ],
        out_specs=(
            pl.BlockSpec(memory_space=pltpu.VMEM),
            pl.BlockSpec(memory_space=pltpu.VMEM),
        ),
        scratch_shapes=[
            pltpu.VMEM((128, R0), jnp.float32),
            pltpu.VMEM((128, R1), jnp.float32),
            pltpu.VMEM((R1, 1), jnp.float32),
        ],
        interpret=interpret,
    )(dt2, sdf2, dirs_t, grads_t, car, beta,
      jnp.full((1,), 1e-06, dtype=jnp.float32))


def _finish_kernel(excl_ref, base_ref, alpha_ref, t_ref, w_ref):
    within = excl_ref[...] - base_ref[...]
    t = jnp.exp(within)
    t_ref[...] = t
    w_ref[...] = alpha_ref[...] * t


def _finish(excl2, base2, alpha2, interpret=False):
    spec = pl.BlockSpec((512, 128), lambda i: (i, 0))
    return pl.pallas_call(
        _finish_kernel,
        out_shape=(
            jax.ShapeDtypeStruct((R0, 128), jnp.float32),
            jax.ShapeDtypeStruct((R0, 128), jnp.float32),
        ),
        grid=(R0 // 512,),
        in_specs=[spec, spec, spec],
        out_specs=(spec, spec),
        interpret=interpret,
    )(excl2, base2, alpha2)


def _expand_body(excl_hbm, cu_hbm, base_hbm,
                 cu_v, g_v, z_v, zm_v, ic_v, base_v, sem_g):
    wid = lax.axis_index("s") * 2 + lax.axis_index("c")
    lo = wid * CHUNK
    nblk = CHUNK // L

    pltpu.sync_copy(cu_hbm, cu_v)

    zeros16 = jnp.zeros((L,), jnp.int32)

    def zbody(k, _):
        z_v[pl.ds(k * L, L)] = zeros16
        return 0

    lax.fori_loop(0, nblk, zbody, 0, unroll=4)

    copies = [
        pltpu.async_copy(excl_hbm.at[cu_v.at[pl.ds(j * 128, 128)]],
                         g_v.at[pl.ds(j * 128, 128)], sem_g)
        for j in range(NRAYS // 128)
    ]

    iota16 = lax.iota(jnp.int32, L)

    def sbody(j, cnt):
        cu16 = cu_v[pl.ds(j * L, L)]
        cnt = cnt + jnp.where(cu16 <= lo, 1, 0).astype(jnp.int32)
        idx16 = cu16 - lo
        rid16 = iota16 + j * L
        mask = (idx16 >= 0) & (idx16 < CHUNK)
        plsc.store_scatter(z_v, [idx16], rid16, mask=mask)
        return cnt

    cnt = lax.fori_loop(0, NRAYS // L, sbody, jnp.zeros((L,), jnp.int32),
                        unroll=4)
    r_lo = jnp.broadcast_to(jnp.sum(cnt) - 1, (L,))

    def abody(k, _):
        zm_v[pl.ds(k * L, L)] = plsc.cummax(z_v[pl.ds(k * L, L)])
        return 0

    lax.fori_loop(0, nblk, abody, 0, unroll=1)

    ic_v[pl.ds(0, L)] = r_lo

    def bbody(kk, carry):
        idx = (iota16 + kk * L) * L + (L - 1)
        bm16 = plsc.load_gather(zm_v, [idx])
        c16 = jnp.maximum(plsc.cummax(bm16), carry)
        ic_v[pl.ds(L + kk * L, L)] = c16
        return jnp.broadcast_to(jnp.max(c16), (L,))

    lax.fori_loop(0, nblk // L, bbody, r_lo, unroll=1)

    for c in copies:
        c.wait()

    def cbody(k, _):
        m = zm_v[pl.ds(k * L, L)]
        prev = plsc.load_gather(ic_v, [jnp.broadcast_to(k + L - 1, (L,))])
        base_v[pl.ds(k * L, L)] = plsc.load_gather(g_v, [jnp.maximum(m, prev)])
        return 0

    lax.fori_loop(0, nblk, cbody, 0, unroll=1)

    pltpu.sync_copy(base_v, base_hbm.at[pl.ds(lo, CHUNK)])


def _base_from_starts(excl_flat, cu_seqlens):
    cu_starts = lax.slice(cu_seqlens, (0,), (NRAYS,))
    return pl.kernel(
        _expand_body,
        out_type=jax.ShapeDtypeStruct((N,), jnp.float32),
        mesh=plsc.VectorSubcoreMesh(core_axis_name="c", subcore_axis_name="s"),
        compiler_params=pltpu.CompilerParams(needs_layout_passes=False),
        scratch_types=[
            pltpu.VMEM((NRAYS,), jnp.int32),
            pltpu.VMEM((NRAYS,), jnp.float32),
            pltpu.VMEM((CHUNK,), jnp.int32),
            pltpu.VMEM((CHUNK,), jnp.int32),
            pltpu.VMEM((CHUNK // L + L,), jnp.int32),
            pltpu.VMEM((CHUNK,), jnp.float32),
            pltpu.SemaphoreType.DMA,
        ],
    )(excl_flat, cu_starts)


def kernel(samples_dt, samples_dirs, sdf, gradients, cos_anneal_ratio,
           logistic_beta, cu_seqlens, interpret=False):
    dt2 = samples_dt.reshape(R0, 128)
    sdf2 = sdf.reshape(R0, 128)
    dirs_t = samples_dirs.T.reshape(3, R0, 128)
    grads_t = gradients.T.reshape(3, R0, 128)
    alpha2, excl2 = _alpha_scan(dt2, sdf2, dirs_t, grads_t,
                                cos_anneal_ratio, logistic_beta,
                                interpret=interpret)
    base2 = _base_from_starts(excl2.reshape(N), cu_seqlens).reshape(R0, 128)
    t2, w2 = _finish(excl2, base2, alpha2, interpret=interpret)
    return (w2.reshape(N, 1), t2.reshape(N, 1), alpha2.reshape(N, 1))
